# baseline (speedup 1.0000x reference)
import functools

import jax
import jax.numpy as jnp
from jax import lax
from jax.experimental import pallas as pl
from jax.experimental.pallas import tpu as pltpu
from jax.experimental.pallas import tpu_sc as plsc

B = 16384
D = 128
S = 10
NC, NS = 2, 16
NW = NC * NS
C = 32
RPW = B // NW
KCH = RPW // C
LANES = 16

_sc_mesh = plsc.VectorSubcoreMesh(core_axis_name="c", subcore_axis_name="s")


@functools.partial(
    pl.kernel,
    out_type=(
        jax.ShapeDtypeStruct((B, D), jnp.float32),
        jax.ShapeDtypeStruct((B, D), jnp.float32),
    ),
    mesh=_sc_mesh,
    scratch_types=[
        pltpu.VMEM((C,), jnp.int32),
        pltpu.VMEM((C,), jnp.int32),
        pltpu.VMEM((C * S,), jnp.int32),
        pltpu.VMEM((C * S,), jnp.int32),
        pltpu.VMEM((C, D), jnp.float32),
        pltpu.VMEM((C, D), jnp.float32),
        pltpu.VMEM((C * S, D), jnp.float32),
        pltpu.VMEM((C * S, D), jnp.float32),
        pltpu.VMEM((C, D), jnp.float32),
        pltpu.VMEM((C, D), jnp.float32),
        pltpu.SemaphoreType.DMA,
        pltpu.SemaphoreType.DMA,
        pltpu.SemaphoreType.DMA,
        pltpu.SemaphoreType.DMA,
        pltpu.SemaphoreType.DMA,
        pltpu.SemaphoreType.DMA,
    ],
)
def _sc_gather_sum(nodes_hbm, nidx_hbm, feat_hbm, self_out, nsum_out,
                   si0, si1, ni0, ni1, sb0, sb1, nb0, nb1, acc0, acc1,
                   g0, g1, s0, s1, a0, a1):
    wid = lax.axis_index("s") * NC + lax.axis_index("c")
    sidx = [si0, si1]
    nidx = [ni0, ni1]
    sbuf = [sb0, sb1]
    nbuf = [nb0, nb1]
    acc = [acc0, acc1]
    gsem = [g0, g1]
    ssem = [s0, s1]
    asem = [a0, a1]

    def issue_chunk(k, b):
        base = (wid * KCH + k) * C
        pltpu.sync_copy(nodes_hbm.at[pl.ds(base, C)], sidx[b])
        pltpu.sync_copy(nidx_hbm.at[pl.ds(base * S, C * S)], nidx[b])
        copies = [pltpu.async_copy(feat_hbm.at[sidx[b]], sbuf[b], gsem[b])]
        for off in range(0, C * S, 32):
            n = min(32, C * S - off)
            copies.append(
                pltpu.async_copy(
                    feat_hbm.at[nidx[b].at[pl.ds(off, n)]],
                    nbuf[b].at[pl.ds(off, n)],
                    gsem[b],
                )
            )
        return copies

    def accumulate(b):
        src = nbuf[b]
        dst = acc[b]

        def row_body(r, carry):
            for l in range(D // LANES):
                sl = pl.ds(l * LANES, LANES)
                v = src[r * S, sl]
                for j in range(1, S):
                    v = v + src[r * S + j, sl]
                dst[r, sl] = v
            return carry

        lax.fori_loop(0, C, row_body, 0, unroll=False)

    pend_gather = [None, None]
    pend_out = [None, None]

    pend_gather[0] = issue_chunk(0, 0)
    for k in range(KCH):
        b = k % 2
        nb = 1 - b
        if k + 1 < KCH:
            if pend_out[nb] is not None:
                for cp in pend_out[nb]:
                    cp.wait()
                pend_out[nb] = None
            pend_gather[nb] = issue_chunk(k + 1, nb)
        for cp in pend_gather[b]:
            cp.wait()
        base = (wid * KCH + k) * C
        sd = pltpu.async_copy(sbuf[b], self_out.at[pl.ds(base, C)], ssem[b])
        accumulate(b)
        ad = pltpu.async_copy(acc[b], nsum_out.at[pl.ds(base, C)], asem[b])
        pend_out[b] = (sd, ad)

    for b in range(2):
        if pend_out[b] is not None:
            for cp in pend_out[b]:
                cp.wait()


def _tc_body(self_ref, nsum_ref, ws_ref, wn_ref, out_ref):
    z = lax.dot_general(
        ws_ref[...], self_ref[...], (((1,), (1,)), ((), ())),
        preferred_element_type=jnp.float32,
    )
    z += lax.dot_general(
        wn_ref[...], nsum_ref[...], (((1,), (1,)), ((), ())),
        preferred_element_type=jnp.float32,
    )
    out_ref[...] = jnp.maximum(z, 0.0)


_BT = 4096


@jax.jit
def kernel(nodes, neigh_idx, features, weight):
    nodes = nodes.astype(jnp.int32)
    neigh_idx = neigh_idx.astype(jnp.int32).reshape(B * S)

    self_feats, nsum = _sc_gather_sum(nodes, neigh_idx, features)

    w_self = weight[:, :D]
    w_neigh = weight[:, D:] * (1.0 / S)

    out = pl.pallas_call(
        _tc_body,
        grid=(B // _BT,),
        in_specs=[
            pl.BlockSpec((_BT, D), lambda i: (i, 0)),
            pl.BlockSpec((_BT, D), lambda i: (i, 0)),
            pl.BlockSpec((D, D), lambda i: (0, 0)),
            pl.BlockSpec((D, D), lambda i: (0, 0)),
        ],
        out_specs=pl.BlockSpec((D, _BT), lambda i: (0, i)),
        out_shape=jax.ShapeDtypeStruct((D, B), jnp.float32),
    )(self_feats, nsum, w_self, w_neigh)
    return out


# speedup vs baseline: 1.4947x; 1.4947x over previous
import functools

import jax
import jax.numpy as jnp
from jax import lax
from jax.experimental import pallas as pl
from jax.experimental.pallas import tpu as pltpu
from jax.experimental.pallas import tpu_sc as plsc

B = 16384
D = 128
S = 10
R = S + 1
NC, NS = 2, 16
NW = NC * NS
C = 32
RPW = B // NW
KCH = RPW // C
NCHUNK = B // C
LANES = 16

_sc_mesh = plsc.VectorSubcoreMesh(core_axis_name="c", subcore_axis_name="s")


@functools.partial(
    pl.kernel,
    out_type=(
        jax.ShapeDtypeStruct((B, D), jnp.float32),
        jax.ShapeDtypeStruct((B, D), jnp.float32),
    ),
    mesh=_sc_mesh,
    scratch_types=[
        pltpu.VMEM((R, C), jnp.int32),
        pltpu.VMEM((R, C), jnp.int32),
        pltpu.VMEM((R * C, D), jnp.float32),
        pltpu.VMEM((R * C, D), jnp.float32),
        pltpu.VMEM((C, D), jnp.float32),
        pltpu.VMEM((C, D), jnp.float32),
        pltpu.SemaphoreType.DMA,
        pltpu.SemaphoreType.DMA,
        pltpu.SemaphoreType.DMA,
        pltpu.SemaphoreType.DMA,
        pltpu.SemaphoreType.DMA,
        pltpu.SemaphoreType.DMA,
    ],
)
def _sc_gather_sum(idx_hbm, feat_hbm, self_out, nsum_out,
                   idx0, idx1, buf0, buf1, acc0, acc1,
                   g0, g1, s0, s1, a0, a1):
    wid = lax.axis_index("s") * NC + lax.axis_index("c")
    idx = [idx0, idx1]
    buf = [buf0, buf1]
    acc = [acc0, acc1]
    gsem = [g0, g1]
    ssem = [s0, s1]
    asem = [a0, a1]

    def issue_chunk(k, b):
        g = wid * KCH + k
        pltpu.sync_copy(idx_hbm.at[g], idx[b])
        return [
            pltpu.async_copy(
                feat_hbm.at[idx[b].at[j]], buf[b].at[pl.ds(j * C, C)], gsem[b]
            )
            for j in range(R)
        ]

    def accumulate(b):
        src = buf[b]
        dst = acc[b]

        def row_body(r, carry):
            for l in range(D // LANES):
                sl = pl.ds(l * LANES, LANES)
                v = src[C + r, sl]
                for j in range(2, R):
                    v = v + src[j * C + r, sl]
                dst[r, sl] = v
            return carry

        lax.fori_loop(0, C, row_body, 0, unroll=False)

    pend_gather = [None, None]
    pend_out = [None, None]

    pend_gather[0] = issue_chunk(0, 0)
    for k in range(KCH):
        b = k % 2
        nb = 1 - b
        if k + 1 < KCH:
            if pend_out[nb] is not None:
                for cp in pend_out[nb]:
                    cp.wait()
                pend_out[nb] = None
            pend_gather[nb] = issue_chunk(k + 1, nb)
        for cp in pend_gather[b]:
            cp.wait()
        base = (wid * KCH + k) * C
        sd = pltpu.async_copy(
            buf[b].at[pl.ds(0, C)], self_out.at[pl.ds(base, C)], ssem[b]
        )
        accumulate(b)
        ad = pltpu.async_copy(acc[b], nsum_out.at[pl.ds(base, C)], asem[b])
        pend_out[b] = (sd, ad)

    for b in range(2):
        if pend_out[b] is not None:
            for cp in pend_out[b]:
                cp.wait()


def _tc_body(self_ref, nsum_ref, ws_ref, wn_ref, out_ref):
    z = lax.dot_general(
        ws_ref[...], self_ref[...], (((1,), (1,)), ((), ())),
        preferred_element_type=jnp.float32,
    )
    z += lax.dot_general(
        wn_ref[...], nsum_ref[...], (((1,), (1,)), ((), ())),
        preferred_element_type=jnp.float32,
    )
    out_ref[...] = jnp.maximum(z, 0.0)


_BT = 4096


@jax.jit
def kernel(nodes, neigh_idx, features, weight):
    nodes = nodes.astype(jnp.int32)
    neigh_idx = neigh_idx.astype(jnp.int32)
    idx_all = jnp.concatenate([nodes[:, None], neigh_idx], axis=1)
    idx_chunks = idx_all.reshape(NCHUNK, C, R).transpose(0, 2, 1)

    self_feats, nsum = _sc_gather_sum(idx_chunks, features)

    w_self = weight[:, :D]
    w_neigh = weight[:, D:] * (1.0 / S)

    out = pl.pallas_call(
        _tc_body,
        grid=(B // _BT,),
        in_specs=[
            pl.BlockSpec((_BT, D), lambda i: (i, 0)),
            pl.BlockSpec((_BT, D), lambda i: (i, 0)),
            pl.BlockSpec((D, D), lambda i: (0, 0)),
            pl.BlockSpec((D, D), lambda i: (0, 0)),
        ],
        out_specs=pl.BlockSpec((D, _BT), lambda i: (0, i)),
        out_shape=jax.ShapeDtypeStruct((D, B), jnp.float32),
    )(self_feats, nsum, w_self, w_neigh)
    return out


# speedup vs baseline: 1.6051x; 1.0738x over previous
import functools

import jax
import jax.numpy as jnp
from jax import lax
from jax.experimental import pallas as pl
from jax.experimental.pallas import tpu as pltpu
from jax.experimental.pallas import tpu_sc as plsc

B = 16384
D = 128
S = 10
R = S + 1
NC, NS = 2, 16
NW = NC * NS
C = 32
RPW = B // NW
KCH = RPW // C
NCHUNK = B // C
LANES = 16

_sc_mesh = plsc.VectorSubcoreMesh(core_axis_name="c", subcore_axis_name="s")


@functools.partial(
    pl.kernel,
    out_type=(
        jax.ShapeDtypeStruct((B, D), jnp.float32),
        jax.ShapeDtypeStruct((B, D), jnp.float32),
    ),
    mesh=_sc_mesh,
    scratch_types=[
        pltpu.VMEM((R, C), jnp.int32),
        pltpu.VMEM((R, C), jnp.int32),
        pltpu.VMEM((R * C, D), jnp.float32),
        pltpu.VMEM((R * C, D), jnp.float32),
        pltpu.VMEM((C, D), jnp.float32),
        pltpu.VMEM((C, D), jnp.float32),
        pltpu.SemaphoreType.DMA,
        pltpu.SemaphoreType.DMA,
        pltpu.SemaphoreType.DMA,
        pltpu.SemaphoreType.DMA,
        pltpu.SemaphoreType.DMA,
        pltpu.SemaphoreType.DMA,
        pltpu.SemaphoreType.DMA,
        pltpu.SemaphoreType.DMA,
    ],
)
def _sc_gather_sum(idx_hbm, feat_hbm, self_out, nsum_out,
                   idx0, idx1, buf0, buf1, acc0, acc1,
                   g0, g1, s0, s1, a0, a1, i0, i1):
    wid = lax.axis_index("s") * NC + lax.axis_index("c")
    idx = [idx0, idx1]
    buf = [buf0, buf1]
    acc = [acc0, acc1]
    gsem = [g0, g1]
    ssem = [s0, s1]
    asem = [a0, a1]
    isem = [i0, i1]

    def prefetch_idx(k, b):
        g = wid * KCH + k
        return pltpu.async_copy(idx_hbm.at[g], idx[b], isem[b])

    def fire_gathers(b):
        return [
            pltpu.async_copy(
                feat_hbm.at[idx[b].at[j]], buf[b].at[pl.ds(j * C, C)], gsem[b]
            )
            for j in range(R)
        ]

    def accumulate(b):
        src = buf[b]
        dst = acc[b]

        def row_body(r, carry):
            for l in range(D // LANES):
                sl = pl.ds(l * LANES, LANES)
                v = src[C + r, sl]
                for j in range(2, R):
                    v = v + src[j * C + r, sl]
                dst[r, sl] = v
            return carry

        lax.fori_loop(0, C, row_body, 0, unroll=False)

    pend_gather = [None, None]
    pend_out = [None, None]

    prefetch_idx(0, 0).wait()
    pend_gather[0] = fire_gathers(0)
    pend_idx = prefetch_idx(1, 1) if KCH > 1 else None
    for k in range(KCH):
        b = k % 2
        nb = 1 - b
        if k + 1 < KCH:
            if pend_out[nb] is not None:
                for cp in pend_out[nb]:
                    cp.wait()
                pend_out[nb] = None
            pend_idx.wait()
            pend_gather[nb] = fire_gathers(nb)
        for cp in pend_gather[b]:
            cp.wait()
        if k + 2 < KCH:
            pend_idx = prefetch_idx(k + 2, b)
        base = (wid * KCH + k) * C
        sd = pltpu.async_copy(
            buf[b].at[pl.ds(0, C)], self_out.at[pl.ds(base, C)], ssem[b]
        )
        accumulate(b)
        ad = pltpu.async_copy(acc[b], nsum_out.at[pl.ds(base, C)], asem[b])
        pend_out[b] = (sd, ad)

    for b in range(2):
        if pend_out[b] is not None:
            for cp in pend_out[b]:
                cp.wait()


def _tc_body(self_ref, nsum_ref, ws_ref, wn_ref, out_ref):
    z = lax.dot_general(
        ws_ref[...], self_ref[...], (((1,), (1,)), ((), ())),
        preferred_element_type=jnp.float32,
    )
    z += lax.dot_general(
        wn_ref[...], nsum_ref[...], (((1,), (1,)), ((), ())),
        preferred_element_type=jnp.float32,
    )
    out_ref[...] = jnp.maximum(z, 0.0)


_BT = 4096


@jax.jit
def kernel(nodes, neigh_idx, features, weight):
    nodes = nodes.astype(jnp.int32)
    neigh_idx = neigh_idx.astype(jnp.int32)
    idx_all = jnp.concatenate([nodes[:, None], neigh_idx], axis=1)
    idx_chunks = idx_all.reshape(NCHUNK, C, R).transpose(0, 2, 1)

    self_feats, nsum = _sc_gather_sum(idx_chunks, features)

    w_self = weight[:, :D]
    w_neigh = weight[:, D:] * (1.0 / S)

    out = pl.pallas_call(
        _tc_body,
        grid=(B // _BT,),
        in_specs=[
            pl.BlockSpec((_BT, D), lambda i: (i, 0)),
            pl.BlockSpec((_BT, D), lambda i: (i, 0)),
            pl.BlockSpec((D, D), lambda i: (0, 0)),
            pl.BlockSpec((D, D), lambda i: (0, 0)),
        ],
        out_specs=pl.BlockSpec((D, _BT), lambda i: (0, i)),
        out_shape=jax.ShapeDtypeStruct((D, B), jnp.float32),
    )(self_feats, nsum, w_self, w_neigh)
    return out


# speedup vs baseline: 1.6333x; 1.0176x over previous
import functools

import jax
import jax.numpy as jnp
from jax import lax
from jax.experimental import pallas as pl
from jax.experimental.pallas import tpu as pltpu
from jax.experimental.pallas import tpu_sc as plsc

B = 16384
D = 128
S = 10
NC, NS = 2, 16
NW = NC * NS
C = 32
RPW = B // NW
KCH = RPW // C
NCHUNK = B // C
LANES = 16

_sc_mesh = plsc.VectorSubcoreMesh(core_axis_name="c", subcore_axis_name="s")


@functools.partial(
    pl.kernel,
    out_type=(
        jax.ShapeDtypeStruct((B, D), jnp.float32),
        jax.ShapeDtypeStruct((B, D), jnp.float32),
    ),
    mesh=_sc_mesh,
    scratch_types=[
        pltpu.VMEM((C,), jnp.int32),
        pltpu.VMEM((C,), jnp.int32),
        pltpu.VMEM((S, C), jnp.int32),
        pltpu.VMEM((S, C), jnp.int32),
        pltpu.VMEM((C, D), jnp.float32),
        pltpu.VMEM((C, D), jnp.float32),
        pltpu.VMEM((S * C, D), jnp.float32),
        pltpu.VMEM((S * C, D), jnp.float32),
        pltpu.VMEM((C, D), jnp.float32),
        pltpu.VMEM((C, D), jnp.float32),
        pltpu.SemaphoreType.DMA,
        pltpu.SemaphoreType.DMA,
        pltpu.SemaphoreType.DMA,
        pltpu.SemaphoreType.DMA,
        pltpu.SemaphoreType.DMA,
        pltpu.SemaphoreType.DMA,
        pltpu.SemaphoreType.DMA,
        pltpu.SemaphoreType.DMA,
    ],
)
def _sc_gather_sum(nodes_hbm, nidx_hbm, feat_hbm, self_out, nsum_out,
                   si0, si1, ni0, ni1, sb0, sb1, nb0, nb1, acc0, acc1,
                   g0, g1, s0, s1, a0, a1, i0, i1):
    wid = lax.axis_index("s") * NC + lax.axis_index("c")
    sidx = [si0, si1]
    nidx = [ni0, ni1]
    sbuf = [sb0, sb1]
    nbuf = [nb0, nb1]
    acc = [acc0, acc1]
    gsem = [g0, g1]
    ssem = [s0, s1]
    asem = [a0, a1]
    isem = [i0, i1]

    def prefetch_idx(k, b):
        g = wid * KCH + k
        return [
            pltpu.async_copy(nodes_hbm.at[pl.ds(g * C, C)], sidx[b], isem[b]),
            pltpu.async_copy(nidx_hbm.at[g], nidx[b], isem[b]),
        ]

    def fire_gathers(b):
        copies = [pltpu.async_copy(feat_hbm.at[sidx[b]], sbuf[b], gsem[b])]
        for j in range(S):
            copies.append(
                pltpu.async_copy(
                    feat_hbm.at[nidx[b].at[j]],
                    nbuf[b].at[pl.ds(j * C, C)],
                    gsem[b],
                )
            )
        return copies

    def accumulate(b):
        src = nbuf[b]
        dst = acc[b]

        def row_body(r, carry):
            for l in range(D // LANES):
                sl = pl.ds(l * LANES, LANES)
                v = src[r, sl]
                for j in range(1, S):
                    v = v + src[j * C + r, sl]
                dst[r, sl] = v
            return carry

        lax.fori_loop(0, C, row_body, 0, unroll=False)

    pend_gather = [None, None]
    pend_out = [None, None]

    for cp in prefetch_idx(0, 0):
        cp.wait()
    pend_gather[0] = fire_gathers(0)
    pend_idx = prefetch_idx(1, 1) if KCH > 1 else None
    for k in range(KCH):
        b = k % 2
        nb = 1 - b
        if k + 1 < KCH:
            if pend_out[nb] is not None:
                for cp in pend_out[nb]:
                    cp.wait()
                pend_out[nb] = None
            for cp in pend_idx:
                cp.wait()
            pend_gather[nb] = fire_gathers(nb)
        for cp in pend_gather[b]:
            cp.wait()
        if k + 2 < KCH:
            pend_idx = prefetch_idx(k + 2, b)
        base = (wid * KCH + k) * C
        sd = pltpu.async_copy(sbuf[b], self_out.at[pl.ds(base, C)], ssem[b])
        accumulate(b)
        ad = pltpu.async_copy(acc[b], nsum_out.at[pl.ds(base, C)], asem[b])
        pend_out[b] = (sd, ad)

    for b in range(2):
        if pend_out[b] is not None:
            for cp in pend_out[b]:
                cp.wait()


def _tc_body(self_ref, nsum_ref, ws_ref, wn_ref, out_ref):
    z = lax.dot_general(
        ws_ref[...], self_ref[...], (((1,), (1,)), ((), ())),
        preferred_element_type=jnp.float32,
    )
    z += lax.dot_general(
        wn_ref[...], nsum_ref[...], (((1,), (1,)), ((), ())),
        preferred_element_type=jnp.float32,
    )
    out_ref[...] = jnp.maximum(z, 0.0)


_BT = 8192


@jax.jit
def kernel(nodes, neigh_idx, features, weight):
    nodes = nodes.astype(jnp.int32)
    neigh_idx = neigh_idx.astype(jnp.int32)
    nidx_chunks = neigh_idx.reshape(NCHUNK, C, S).transpose(0, 2, 1)

    self_feats, nsum = _sc_gather_sum(nodes, nidx_chunks, features)

    w_self = weight[:, :D]
    w_neigh = weight[:, D:] * (1.0 / S)

    out = pl.pallas_call(
        _tc_body,
        grid=(B // _BT,),
        in_specs=[
            pl.BlockSpec((_BT, D), lambda i: (i, 0)),
            pl.BlockSpec((_BT, D), lambda i: (i, 0)),
            pl.BlockSpec((D, D), lambda i: (0, 0)),
            pl.BlockSpec((D, D), lambda i: (0, 0)),
        ],
        out_specs=pl.BlockSpec((D, _BT), lambda i: (0, i)),
        out_shape=jax.ShapeDtypeStruct((D, B), jnp.float32),
    )(self_feats, nsum, w_self, w_neigh)
    return out


# speedup vs baseline: 1.7120x; 1.0482x over previous
import functools

import jax
import jax.numpy as jnp
from jax import lax
from jax.experimental import pallas as pl
from jax.experimental.pallas import tpu as pltpu
from jax.experimental.pallas import tpu_sc as plsc

B = 16384
D = 128
S = 10
NC, NS = 2, 16
NW = NC * NS
C = 32
RPW = B // NW
KCH = RPW // C
NCHUNK = B // C
LANES = 16

_sc_mesh = plsc.VectorSubcoreMesh(core_axis_name="c", subcore_axis_name="s")


@functools.partial(
    pl.kernel,
    out_type=(
        jax.ShapeDtypeStruct((B, D), jnp.float32),
        jax.ShapeDtypeStruct((B, D), jnp.float32),
    ),
    mesh=_sc_mesh,
    scratch_types=[
        pltpu.VMEM((C,), jnp.int32),
        pltpu.VMEM((C,), jnp.int32),
        pltpu.VMEM((S, C), jnp.int32),
        pltpu.VMEM((S, C), jnp.int32),
        pltpu.VMEM((C, D), jnp.float32),
        pltpu.VMEM((C, D), jnp.float32),
        pltpu.VMEM((C, D), jnp.float32),
        pltpu.VMEM((C, D), jnp.float32),
        pltpu.SemaphoreType.DMA,
        pltpu.SemaphoreType.DMA,
        pltpu.SemaphoreType.DMA,
        pltpu.SemaphoreType.DMA,
        pltpu.SemaphoreType.DMA,
        pltpu.SemaphoreType.DMA,
        pltpu.SemaphoreType.DMA,
        pltpu.SemaphoreType.DMA,
    ],
)
def _sc_gather_sum(nodes_hbm, nidx_hbm, feat_hbm, self_out, nsum_out,
                   si0, si1, ni0, ni1, sb0, sb1, acc0, acc1,
                   g0, g1, s0, s1, a0, a1, i0, i1):
    wid = lax.axis_index("s") * NC + lax.axis_index("c")
    sidx = [si0, si1]
    nidx = [ni0, ni1]
    sbuf = [sb0, sb1]
    acc = [acc0, acc1]
    gsem = [g0, g1]
    ssem = [s0, s1]
    asem = [a0, a1]
    isem = [i0, i1]

    def prefetch_idx(k, b):
        g = wid * KCH + k
        return [
            pltpu.async_copy(nodes_hbm.at[pl.ds(g * C, C)], sidx[b], isem[b]),
            pltpu.async_copy(nidx_hbm.at[g], nidx[b], isem[b]),
        ]

    def fire_gathers(b):
        copies = [pltpu.async_copy(feat_hbm.at[sidx[b]], sbuf[b], gsem[b])]
        for j in range(S):
            copies.append(
                pltpu.async_copy(
                    feat_hbm.at[nidx[b].at[j]],
                    acc[b],
                    gsem[b],
                    add=True,
                )
            )
        return copies

    zv = jnp.zeros((LANES,), jnp.float32)

    def zero_acc(b):
        dst = acc[b]

        def row_body(r, carry):
            for l in range(D // LANES):
                dst[r, pl.ds(l * LANES, LANES)] = zv
            return carry

        lax.fori_loop(0, C, row_body, 0, unroll=False)

    pend_gather = [None, None]
    pend_out = [None, None]

    for cp in prefetch_idx(0, 0):
        cp.wait()
    zero_acc(0)
    pend_gather[0] = fire_gathers(0)
    pend_idx = prefetch_idx(1, 1) if KCH > 1 else None
    for k in range(KCH):
        b = k % 2
        nb = 1 - b
        if k + 1 < KCH:
            if pend_out[nb] is not None:
                for cp in pend_out[nb]:
                    cp.wait()
                pend_out[nb] = None
            zero_acc(nb)
            for cp in pend_idx:
                cp.wait()
            pend_gather[nb] = fire_gathers(nb)
        for cp in pend_gather[b]:
            cp.wait()
        if k + 2 < KCH:
            pend_idx = prefetch_idx(k + 2, b)
        base = (wid * KCH + k) * C
        sd = pltpu.async_copy(sbuf[b], self_out.at[pl.ds(base, C)], ssem[b])
        ad = pltpu.async_copy(acc[b], nsum_out.at[pl.ds(base, C)], asem[b])
        pend_out[b] = (sd, ad)

    for b in range(2):
        if pend_out[b] is not None:
            for cp in pend_out[b]:
                cp.wait()


def _tc_body(self_ref, nsum_ref, ws_ref, wn_ref, out_ref):
    z = lax.dot_general(
        ws_ref[...], self_ref[...], (((1,), (1,)), ((), ())),
        preferred_element_type=jnp.float32,
    )
    z += lax.dot_general(
        wn_ref[...], nsum_ref[...], (((1,), (1,)), ((), ())),
        preferred_element_type=jnp.float32,
    )
    out_ref[...] = jnp.maximum(z, 0.0)


_BT = 8192


@jax.jit
def kernel(nodes, neigh_idx, features, weight):
    nodes = nodes.astype(jnp.int32)
    neigh_idx = neigh_idx.astype(jnp.int32)
    nidx_chunks = neigh_idx.reshape(NCHUNK, C, S).transpose(0, 2, 1)

    self_feats, nsum = _sc_gather_sum(nodes, nidx_chunks, features)

    w_self = weight[:, :D]
    w_neigh = weight[:, D:] * (1.0 / S)

    out = pl.pallas_call(
        _tc_body,
        grid=(B // _BT,),
        in_specs=[
            pl.BlockSpec((_BT, D), lambda i: (i, 0)),
            pl.BlockSpec((_BT, D), lambda i: (i, 0)),
            pl.BlockSpec((D, D), lambda i: (0, 0)),
            pl.BlockSpec((D, D), lambda i: (0, 0)),
        ],
        out_specs=pl.BlockSpec((D, _BT), lambda i: (0, i)),
        out_shape=jax.ShapeDtypeStruct((D, B), jnp.float32),
    )(self_feats, nsum, w_self, w_neigh)
    return out


# speedup vs baseline: 1.8411x; 1.0754x over previous
import functools

import jax
import jax.numpy as jnp
from jax import lax
from jax.experimental import pallas as pl
from jax.experimental.pallas import tpu as pltpu
from jax.experimental.pallas import tpu_sc as plsc

B = 16384
D = 128
S = 10
NC, NS = 2, 16
NW = NC * NS
C = 64
RPW = B // NW
KCH = RPW // C
NCHUNK = B // C
LANES = 16

_sc_mesh = plsc.VectorSubcoreMesh(core_axis_name="c", subcore_axis_name="s")


@functools.partial(
    pl.kernel,
    out_type=(
        jax.ShapeDtypeStruct((B, D), jnp.float32),
        jax.ShapeDtypeStruct((B, D), jnp.float32),
    ),
    mesh=_sc_mesh,
    scratch_types=[
        pltpu.VMEM((C,), jnp.int32),
        pltpu.VMEM((C,), jnp.int32),
        pltpu.VMEM((S, C), jnp.int32),
        pltpu.VMEM((S, C), jnp.int32),
        pltpu.VMEM((C, D), jnp.float32),
        pltpu.VMEM((C, D), jnp.float32),
        pltpu.VMEM((C, D), jnp.float32),
        pltpu.VMEM((C, D), jnp.float32),
        pltpu.SemaphoreType.DMA,
        pltpu.SemaphoreType.DMA,
        pltpu.SemaphoreType.DMA,
        pltpu.SemaphoreType.DMA,
        pltpu.SemaphoreType.DMA,
        pltpu.SemaphoreType.DMA,
        pltpu.SemaphoreType.DMA,
        pltpu.SemaphoreType.DMA,
    ],
)
def _sc_gather_sum(nodes_hbm, nidx_hbm, feat_hbm, self_out, nsum_out,
                   si0, si1, ni0, ni1, sb0, sb1, acc0, acc1,
                   g0, g1, s0, s1, a0, a1, i0, i1):
    wid = lax.axis_index("s") * NC + lax.axis_index("c")
    sidx = [si0, si1]
    nidx = [ni0, ni1]
    sbuf = [sb0, sb1]
    acc = [acc0, acc1]
    gsem = [g0, g1]
    ssem = [s0, s1]
    asem = [a0, a1]
    isem = [i0, i1]

    def prefetch_idx(k, b):
        g = wid * KCH + k
        return [
            pltpu.async_copy(nodes_hbm.at[pl.ds(g * C, C)], sidx[b], isem[b]),
            pltpu.async_copy(nidx_hbm.at[g], nidx[b], isem[b]),
        ]

    def fire_gathers(b):
        copies = [pltpu.async_copy(feat_hbm.at[sidx[b]], sbuf[b], gsem[b])]
        for j in range(S):
            copies.append(
                pltpu.async_copy(
                    feat_hbm.at[nidx[b].at[j]],
                    acc[b],
                    gsem[b],
                    add=True,
                )
            )
        return copies

    zv = jnp.zeros((LANES,), jnp.float32)

    def zero_acc(b):
        dst = acc[b]

        def row_body(r, carry):
            for l in range(D // LANES):
                dst[r, pl.ds(l * LANES, LANES)] = zv
            return carry

        lax.fori_loop(0, C, row_body, 0, unroll=False)

    pend_gather = [None, None]
    pend_out = [None, None]

    for cp in prefetch_idx(0, 0):
        cp.wait()
    zero_acc(0)
    pend_gather[0] = fire_gathers(0)
    pend_idx = prefetch_idx(1, 1) if KCH > 1 else None
    for k in range(KCH):
        b = k % 2
        nb = 1 - b
        if k + 1 < KCH:
            if pend_out[nb] is not None:
                for cp in pend_out[nb]:
                    cp.wait()
                pend_out[nb] = None
            zero_acc(nb)
            for cp in pend_idx:
                cp.wait()
            pend_gather[nb] = fire_gathers(nb)
        for cp in pend_gather[b]:
            cp.wait()
        if k + 2 < KCH:
            pend_idx = prefetch_idx(k + 2, b)
        base = (wid * KCH + k) * C
        sd = pltpu.async_copy(sbuf[b], self_out.at[pl.ds(base, C)], ssem[b])
        ad = pltpu.async_copy(acc[b], nsum_out.at[pl.ds(base, C)], asem[b])
        pend_out[b] = (sd, ad)

    for b in range(2):
        if pend_out[b] is not None:
            for cp in pend_out[b]:
                cp.wait()


def _tc_body(self_ref, nsum_ref, ws_ref, wn_ref, out_ref):
    z = lax.dot_general(
        ws_ref[...], self_ref[...], (((1,), (1,)), ((), ())),
        preferred_element_type=jnp.float32,
    )
    z += lax.dot_general(
        wn_ref[...], nsum_ref[...], (((1,), (1,)), ((), ())),
        preferred_element_type=jnp.float32,
    )
    out_ref[...] = jnp.maximum(z, 0.0)


_BT = 8192


@jax.jit
def kernel(nodes, neigh_idx, features, weight):
    nodes = nodes.astype(jnp.int32)
    neigh_idx = neigh_idx.astype(jnp.int32)
    nidx_chunks = neigh_idx.reshape(NCHUNK, C, S).transpose(0, 2, 1)

    self_feats, nsum = _sc_gather_sum(nodes, nidx_chunks, features)

    w_self = weight[:, :D]
    w_neigh = weight[:, D:] * (1.0 / S)

    out = pl.pallas_call(
        _tc_body,
        grid=(B // _BT,),
        in_specs=[
            pl.BlockSpec((_BT, D), lambda i: (i, 0)),
            pl.BlockSpec((_BT, D), lambda i: (i, 0)),
            pl.BlockSpec((D, D), lambda i: (0, 0)),
            pl.BlockSpec((D, D), lambda i: (0, 0)),
        ],
        out_specs=pl.BlockSpec((D, _BT), lambda i: (0, i)),
        out_shape=jax.ShapeDtypeStruct((D, B), jnp.float32),
    )(self_feats, nsum, w_self, w_neigh)
    return out


# speedup vs baseline: 1.9333x; 1.0501x over previous
import functools

import jax
import jax.numpy as jnp
from jax import lax
from jax.experimental import pallas as pl
from jax.experimental.pallas import tpu as pltpu
from jax.experimental.pallas import tpu_sc as plsc

B = 16384
D = 128
S = 10
NC, NS = 2, 16
NW = NC * NS
C = 128
RPW = B // NW
KCH = RPW // C
NCHUNK = B // C
LANES = 16

_sc_mesh = plsc.VectorSubcoreMesh(core_axis_name="c", subcore_axis_name="s")


@functools.partial(
    pl.kernel,
    out_type=(
        jax.ShapeDtypeStruct((B, D), jnp.float32),
        jax.ShapeDtypeStruct((B, D), jnp.float32),
    ),
    mesh=_sc_mesh,
    scratch_types=[
        pltpu.VMEM((C,), jnp.int32),
        pltpu.VMEM((C,), jnp.int32),
        pltpu.VMEM((S, C), jnp.int32),
        pltpu.VMEM((S, C), jnp.int32),
        pltpu.VMEM((C, D), jnp.float32),
        pltpu.VMEM((C, D), jnp.float32),
        pltpu.VMEM((C, D), jnp.float32),
        pltpu.VMEM((C, D), jnp.float32),
        pltpu.SemaphoreType.DMA,
        pltpu.SemaphoreType.DMA,
        pltpu.SemaphoreType.DMA,
        pltpu.SemaphoreType.DMA,
        pltpu.SemaphoreType.DMA,
        pltpu.SemaphoreType.DMA,
        pltpu.SemaphoreType.DMA,
        pltpu.SemaphoreType.DMA,
    ],
)
def _sc_gather_sum(nodes_hbm, nidx_hbm, feat_hbm, self_out, nsum_out,
                   si0, si1, ni0, ni1, sb0, sb1, acc0, acc1,
                   g0, g1, s0, s1, a0, a1, i0, i1):
    wid = lax.axis_index("s") * NC + lax.axis_index("c")
    sidx = [si0, si1]
    nidx = [ni0, ni1]
    sbuf = [sb0, sb1]
    acc = [acc0, acc1]
    gsem = [g0, g1]
    ssem = [s0, s1]
    asem = [a0, a1]
    isem = [i0, i1]

    def prefetch_idx(k, b):
        g = wid * KCH + k
        return [
            pltpu.async_copy(nodes_hbm.at[pl.ds(g * C, C)], sidx[b], isem[b]),
            pltpu.async_copy(nidx_hbm.at[g], nidx[b], isem[b]),
        ]

    def fire_gathers(b):
        copies = [pltpu.async_copy(feat_hbm.at[sidx[b]], sbuf[b], gsem[b])]
        for j in range(S):
            copies.append(
                pltpu.async_copy(
                    feat_hbm.at[nidx[b].at[j]],
                    acc[b],
                    gsem[b],
                    add=True,
                )
            )
        return copies

    zv = jnp.zeros((LANES,), jnp.float32)

    def zero_acc(b):
        dst = acc[b]

        def row_body(r, carry):
            for l in range(D // LANES):
                dst[r, pl.ds(l * LANES, LANES)] = zv
            return carry

        lax.fori_loop(0, C, row_body, 0, unroll=False)

    pend_gather = [None, None]
    pend_out = [None, None]

    for cp in prefetch_idx(0, 0):
        cp.wait()
    zero_acc(0)
    pend_gather[0] = fire_gathers(0)
    pend_idx = prefetch_idx(1, 1) if KCH > 1 else None
    for k in range(KCH):
        b = k % 2
        nb = 1 - b
        if k + 1 < KCH:
            if pend_out[nb] is not None:
                for cp in pend_out[nb]:
                    cp.wait()
                pend_out[nb] = None
            zero_acc(nb)
            for cp in pend_idx:
                cp.wait()
            pend_gather[nb] = fire_gathers(nb)
        for cp in pend_gather[b]:
            cp.wait()
        if k + 2 < KCH:
            pend_idx = prefetch_idx(k + 2, b)
        base = (wid * KCH + k) * C
        sd = pltpu.async_copy(sbuf[b], self_out.at[pl.ds(base, C)], ssem[b])
        ad = pltpu.async_copy(acc[b], nsum_out.at[pl.ds(base, C)], asem[b])
        pend_out[b] = (sd, ad)

    for b in range(2):
        if pend_out[b] is not None:
            for cp in pend_out[b]:
                cp.wait()


def _tc_body(self_ref, nsum_ref, ws_ref, wn_ref, out_ref):
    z = lax.dot_general(
        ws_ref[...], self_ref[...], (((1,), (1,)), ((), ())),
        preferred_element_type=jnp.float32,
    )
    z += lax.dot_general(
        wn_ref[...], nsum_ref[...], (((1,), (1,)), ((), ())),
        preferred_element_type=jnp.float32,
    )
    out_ref[...] = jnp.maximum(z, 0.0)


_BT = 8192


@jax.jit
def kernel(nodes, neigh_idx, features, weight):
    nodes = nodes.astype(jnp.int32)
    neigh_idx = neigh_idx.astype(jnp.int32)
    nidx_chunks = neigh_idx.reshape(NCHUNK, C, S).transpose(0, 2, 1)

    self_feats, nsum = _sc_gather_sum(nodes, nidx_chunks, features)

    w_self = weight[:, :D]
    w_neigh = weight[:, D:] * (1.0 / S)

    out = pl.pallas_call(
        _tc_body,
        grid=(B // _BT,),
        in_specs=[
            pl.BlockSpec((_BT, D), lambda i: (i, 0)),
            pl.BlockSpec((_BT, D), lambda i: (i, 0)),
            pl.BlockSpec((D, D), lambda i: (0, 0)),
            pl.BlockSpec((D, D), lambda i: (0, 0)),
        ],
        out_specs=pl.BlockSpec((D, _BT), lambda i: (0, i)),
        out_shape=jax.ShapeDtypeStruct((D, B), jnp.float32),
    )(self_feats, nsum, w_self, w_neigh)
    return out
